# baseline (speedup 1.0000x reference)
import functools

import jax
import jax.numpy as jnp
from jax import lax
from jax.experimental import pallas as pl
from jax.experimental.pallas import tpu as pltpu
from jax.experimental.pallas import tpu_sc as plsc

_VOCAB = 1_000_000
_D = 32
_K = 5
_NW = 32
_B = 16384
_N = _B * _K
_BC = _B // _NW
_BN = _N // _NW
_R = 256
_LW = 512
_NBLK = (_VOCAB + _LW - 1) // _LW
_LINES = _NBLK * 128


def _linify(table_t):
    def body(x_ref, o_ref):
        x = x_ref[...]
        o_ref[...] = jnp.concatenate(
            [x[:, 128 * j:128 * (j + 1)].T for j in range(4)], axis=1)

    return pl.pallas_call(
        body,
        grid=(_NBLK,),
        in_specs=[pl.BlockSpec((_D, _LW), lambda i: (0, i))],
        out_specs=pl.BlockSpec((_LW // 4, 128), lambda i: (i, 0)),
        out_shape=jax.ShapeDtypeStruct((_LINES, 128), jnp.float32),
    )(table_t)


def _sc_gather(lines_c, lines_x, cidx, nidx):
    mesh = plsc.VectorSubcoreMesh(core_axis_name="c", subcore_axis_name="s")

    @functools.partial(
        pl.kernel,
        mesh=mesh,
        compiler_params=pltpu.CompilerParams(
            use_tc_tiling_on_sc=False, needs_layout_passes=False),
        out_type=(
            jax.ShapeDtypeStruct((_D * _B,), jnp.float32),
            jax.ShapeDtypeStruct((_D * _N,), jnp.float32),
        ),
        scratch_types=[
            pltpu.VMEM((_BC,), jnp.int32),
            pltpu.VMEM((_BN,), jnp.int32),
            pltpu.VMEM((_BC,), jnp.int32),
            pltpu.VMEM((_BN,), jnp.int32),
            pltpu.VMEM((_BC,), jnp.int32),
            pltpu.VMEM((_BN,), jnp.int32),
            pltpu.VMEM((_R, 128), jnp.float32),
            pltpu.VMEM((_D, _R), jnp.float32),
            pltpu.SemaphoreType.DMA,
        ],
    )
    def gather_kernel(lc_hbm, lx_hbm, ci_hbm, ni_hbm, co_hbm, no_hbm,
                      cidx_v, nidx_v, clid_v, nlid_v, clane_v, nlane_v,
                      lines_v, vals_v, sem):
        wid = lax.axis_index("s") * 2 + lax.axis_index("c")
        iota16 = lax.iota(jnp.int32, 16)
        cbase = wid * _BC
        nbase = wid * _BN

        pltpu.sync_copy(ci_hbm.at[pl.ds(cbase, _BC)], cidx_v)
        pltpu.sync_copy(ni_hbm.at[pl.ds(nbase, _BN)], nidx_v)
        for g in range(_BC // 16):
            sl = pl.ds(16 * g, 16)
            r = cidx_v[sl]
            clid_v[sl] = ((r >> 9) << 7) + (r & 127)
            clane_v[sl] = ((r >> 7) & 3) << 5
        for g in range(_BN // 16):
            sl = pl.ds(16 * g, 16)
            r = nidx_v[sl]
            nlid_v[sl] = ((r >> 9) << 7) + (r & 127)
            nlane_v[sl] = ((r >> 7) & 3) << 5

        def round_body(src_hbm, lid_v, lane_v, out_hbm, out_stride, obase, t):
            pltpu.async_copy(
                src_hbm.at[lid_v.at[pl.ds(t * _R, _R)]], lines_v, sem
            ).wait()
            for c in range(_D):
                for g in range(_R // 16):
                    row16 = iota16 + (16 * g)
                    lane16 = lane_v[pl.ds(t * _R + 16 * g, 16)] + c
                    vals_v[c, pl.ds(16 * g, 16)] = plsc.load_gather(
                        lines_v, [row16, lane16])
            for c in range(_D):
                pltpu.sync_copy(
                    vals_v.at[c],
                    out_hbm.at[pl.ds(c * out_stride + obase + t * _R, _R)])

        for t in range(_BC // _R):
            round_body(lc_hbm, clid_v, clane_v, co_hbm, _B, cbase, t)

        @pl.loop(0, _BN // _R)
        def _(t):
            round_body(lx_hbm, nlid_v, nlane_v, no_hbm, _N, nbase, t)

    return gather_kernel(lines_c, lines_x, cidx, nidx)


def _log_sigmoid(x):
    return jnp.where(x >= 0, -jnp.log1p(jnp.exp(-x)), x - jnp.log1p(jnp.exp(x)))


def _tc_loss(pos2d, cep, nep):
    def loss_body(pos_ref, ce_ref, ne_ref, out_ref):
        pos_acc = jnp.sum(_log_sigmoid(pos_ref[...]))
        neg_acc = jnp.float32(0.0)
        rows_b = _B // 128
        rows_n = _N // 128
        for k in range(_K):
            s = jnp.zeros((rows_b, 128), jnp.float32)
            for c in range(_D):
                ce = ce_ref[pl.ds(c * rows_b, rows_b), :]
                ne = ne_ref[pl.ds(c * rows_n + k * rows_b, rows_b), :]
                s += ce * ne
            neg_acc += jnp.sum(_log_sigmoid(-s))
        out_ref[0, 0] = -pos_acc / _B - neg_acc / _N

    return pl.pallas_call(
        loss_body,
        out_shape=jax.ShapeDtypeStruct((1, 1), jnp.float32),
        out_specs=pl.BlockSpec(memory_space=pltpu.SMEM),
    )(pos2d, cep, nep)


def kernel(pos_scores, center_words, center_table, context_table):
    batch = pos_scores.shape[0]
    cidx = center_words.astype(jnp.int32)
    neg_words = jax.random.randint(
        jax.random.key(42), (batch, _K), 0, _VOCAB)
    nidx = neg_words.T.reshape(-1).astype(jnp.int32)
    lines_c = _linify(center_table.T)
    lines_x = _linify(context_table.T)
    ce_flat, ne_flat = _sc_gather(lines_c, lines_x, cidx, nidx)
    pos2d = pos_scores.reshape(128, 128)
    cep = ce_flat.reshape(_D * _B // 128, 128)
    nep = ne_flat.reshape(_D * _N // 128, 128)
    loss = _tc_loss(pos2d, cep, nep)
    return loss.reshape(())


# speedup vs baseline: 3.6170x; 3.6170x over previous
import functools

import jax
import jax.numpy as jnp
from jax import lax
from jax.experimental import pallas as pl
from jax.experimental.pallas import tpu as pltpu
from jax.experimental.pallas import tpu_sc as plsc

_VOCAB = 1_000_000
_D = 32
_K = 5
_NW = 32
_B = 16384
_N = _B * _K
_BC = _B // _NW
_BN = _N // _NW
_R = 256
_LW = 8192
_NBLK = (_VOCAB + _LW - 1) // _LW
_LINES = _NBLK * (_LW // 4)


def _linify(table_t):
    def body(x_ref, o_ref):
        x = x_ref[...]
        groups = []
        for g in range(_LW // 512):
            groups.append(jnp.concatenate(
                [x[:, 512 * g + 128 * j:512 * g + 128 * (j + 1)].T
                 for j in range(4)], axis=1))
        o_ref[...] = jnp.concatenate(groups, axis=0)

    return pl.pallas_call(
        body,
        grid=(_NBLK,),
        in_specs=[pl.BlockSpec((_D, _LW), lambda i: (0, i))],
        out_specs=pl.BlockSpec((_LW // 4, 128), lambda i: (i, 0)),
        out_shape=jax.ShapeDtypeStruct((_LINES, 128), jnp.float32),
    )(table_t)


def _sc_gather(lines_c, lines_x, cidx, nidx):
    mesh = plsc.VectorSubcoreMesh(core_axis_name="c", subcore_axis_name="s")

    @functools.partial(
        pl.kernel,
        mesh=mesh,
        compiler_params=pltpu.CompilerParams(
            use_tc_tiling_on_sc=False, needs_layout_passes=False),
        out_type=(
            jax.ShapeDtypeStruct((_D * _B,), jnp.float32),
            jax.ShapeDtypeStruct((_D * _N,), jnp.float32),
        ),
        scratch_types=[
            pltpu.VMEM((_BC,), jnp.int32),
            pltpu.VMEM((_BN,), jnp.int32),
            pltpu.VMEM((_BC,), jnp.int32),
            pltpu.VMEM((_BN,), jnp.int32),
            pltpu.VMEM((_BC,), jnp.int32),
            pltpu.VMEM((_BN,), jnp.int32),
            pltpu.VMEM((_R, 128), jnp.float32),
            pltpu.VMEM((_D, _R), jnp.float32),
            pltpu.SemaphoreType.DMA,
        ],
    )
    def gather_kernel(lc_hbm, lx_hbm, ci_hbm, ni_hbm, co_hbm, no_hbm,
                      cidx_v, nidx_v, clid_v, nlid_v, clane_v, nlane_v,
                      lines_v, vals_v, sem):
        wid = lax.axis_index("s") * 2 + lax.axis_index("c")
        iota16 = lax.iota(jnp.int32, 16)
        cbase = wid * _BC
        nbase = wid * _BN

        pltpu.sync_copy(ci_hbm.at[pl.ds(cbase, _BC)], cidx_v)
        pltpu.sync_copy(ni_hbm.at[pl.ds(nbase, _BN)], nidx_v)
        for g in range(_BC // 16):
            sl = pl.ds(16 * g, 16)
            r = cidx_v[sl]
            clid_v[sl] = ((r >> 9) << 7) + (r & 127)
            clane_v[sl] = ((r >> 7) & 3) << 5
        for g in range(_BN // 16):
            sl = pl.ds(16 * g, 16)
            r = nidx_v[sl]
            nlid_v[sl] = ((r >> 9) << 7) + (r & 127)
            nlane_v[sl] = ((r >> 7) & 3) << 5

        def round_body(src_hbm, lid_v, lane_v, out_hbm, out_stride, obase, t):
            pltpu.async_copy(
                src_hbm.at[lid_v.at[pl.ds(t * _R, _R)]], lines_v, sem
            ).wait()
            for c in range(_D):
                for g in range(_R // 16):
                    row16 = iota16 + (16 * g)
                    lane16 = lane_v[pl.ds(t * _R + 16 * g, 16)] + c
                    vals_v[c, pl.ds(16 * g, 16)] = plsc.load_gather(
                        lines_v, [row16, lane16])
            for c in range(_D):
                pltpu.sync_copy(
                    vals_v.at[c],
                    out_hbm.at[pl.ds(c * out_stride + obase + t * _R, _R)])

        for t in range(_BC // _R):
            round_body(lc_hbm, clid_v, clane_v, co_hbm, _B, cbase, t)

        @pl.loop(0, _BN // _R)
        def _(t):
            round_body(lx_hbm, nlid_v, nlane_v, no_hbm, _N, nbase, t)

    return gather_kernel(lines_c, lines_x, cidx, nidx)


def _log_sigmoid(x):
    return jnp.where(x >= 0, -jnp.log1p(jnp.exp(-x)), x - jnp.log1p(jnp.exp(x)))


def _tc_loss(pos2d, cep, nep):
    def loss_body(pos_ref, ce_ref, ne_ref, out_ref):
        pos_acc = jnp.sum(_log_sigmoid(pos_ref[...]))
        neg_acc = jnp.float32(0.0)
        rows_b = _B // 128
        rows_n = _N // 128
        for k in range(_K):
            s = jnp.zeros((rows_b, 128), jnp.float32)
            for c in range(_D):
                ce = ce_ref[pl.ds(c * rows_b, rows_b), :]
                ne = ne_ref[pl.ds(c * rows_n + k * rows_b, rows_b), :]
                s += ce * ne
            neg_acc += jnp.sum(_log_sigmoid(-s))
        out_ref[0, 0] = -pos_acc / _B - neg_acc / _N

    return pl.pallas_call(
        loss_body,
        out_shape=jax.ShapeDtypeStruct((1, 1), jnp.float32),
        out_specs=pl.BlockSpec(memory_space=pltpu.SMEM),
    )(pos2d, cep, nep)


def kernel(pos_scores, center_words, center_table, context_table):
    batch = pos_scores.shape[0]
    cidx = center_words.astype(jnp.int32)
    neg_words = jax.random.randint(
        jax.random.key(42), (batch, _K), 0, _VOCAB)
    nidx = neg_words.T.reshape(-1).astype(jnp.int32)
    lines_c = _linify(center_table.T)
    lines_x = _linify(context_table.T)
    ce_flat, ne_flat = _sc_gather(lines_c, lines_x, cidx, nidx)
    pos2d = pos_scores.reshape(128, 128)
    cep = ce_flat.reshape(_D * _B // 128, 128)
    nep = ne_flat.reshape(_D * _N // 128, 128)
    loss = _tc_loss(pos2d, cep, nep)
    return loss.reshape(())


# speedup vs baseline: 5.3492x; 1.4789x over previous
import functools

import jax
import jax.numpy as jnp
from jax import lax
from jax.experimental import pallas as pl
from jax.experimental.pallas import tpu as pltpu
from jax.experimental.pallas import tpu_sc as plsc

_VOCAB = 1_000_000
_D = 32
_K = 5
_NW = 32
_B = 16384
_N = _B * _K
_BC = _B // _NW
_BN = _N // _NW
_R = 256
_LW = 8192
_NBLK = (_VOCAB + _LW - 1) // _LW
_LINES = _NBLK * (_LW // 4)


def _linify(table_t):
    def body(x_ref, o_ref):
        x = x_ref[...]
        groups = []
        for g in range(_LW // 512):
            stacked = jnp.concatenate(
                [x[:, 512 * g + 128 * j:512 * g + 128 * (j + 1)]
                 for j in range(4)], axis=0)
            groups.append(stacked.T)
        o_ref[...] = jnp.concatenate(groups, axis=0)

    return pl.pallas_call(
        body,
        grid=(_NBLK,),
        in_specs=[pl.BlockSpec((_D, _LW), lambda i: (0, i))],
        out_specs=pl.BlockSpec((_LW // 4, 128), lambda i: (i, 0)),
        out_shape=jax.ShapeDtypeStruct((_LINES, 128), jnp.float32),
        compiler_params=pltpu.CompilerParams(
            dimension_semantics=("parallel",)),
    )(table_t)


def _sc_gather(lines_c, lines_x, cidx, nidx):
    mesh = plsc.VectorSubcoreMesh(core_axis_name="c", subcore_axis_name="s")

    @functools.partial(
        pl.kernel,
        mesh=mesh,
        compiler_params=pltpu.CompilerParams(
            use_tc_tiling_on_sc=False, needs_layout_passes=False),
        out_type=(
            jax.ShapeDtypeStruct((_D * _B,), jnp.float32),
            jax.ShapeDtypeStruct((_D * _N,), jnp.float32),
        ),
        scratch_types=[
            pltpu.VMEM((_BC,), jnp.int32),
            pltpu.VMEM((_BN,), jnp.int32),
            pltpu.VMEM((_BC,), jnp.int32),
            pltpu.VMEM((_BN,), jnp.int32),
            pltpu.VMEM((_BC,), jnp.int32),
            pltpu.VMEM((_BN,), jnp.int32),
            pltpu.VMEM((_R, 128), jnp.float32),
            pltpu.VMEM((_D, _R), jnp.float32),
            pltpu.SemaphoreType.DMA,
        ],
    )
    def gather_kernel(lc_hbm, lx_hbm, ci_hbm, ni_hbm, co_hbm, no_hbm,
                      cidx_v, nidx_v, clid_v, nlid_v, clane_v, nlane_v,
                      lines_v, vals_v, sem):
        wid = lax.axis_index("s") * 2 + lax.axis_index("c")
        iota16 = lax.iota(jnp.int32, 16)
        cbase = wid * _BC
        nbase = wid * _BN

        pltpu.sync_copy(ci_hbm.at[pl.ds(cbase, _BC)], cidx_v)
        pltpu.sync_copy(ni_hbm.at[pl.ds(nbase, _BN)], nidx_v)
        for g in range(_BC // 16):
            sl = pl.ds(16 * g, 16)
            r = cidx_v[sl]
            clid_v[sl] = ((r >> 9) << 7) + (r & 127)
            clane_v[sl] = ((r >> 7) & 3) << 5
        for g in range(_BN // 16):
            sl = pl.ds(16 * g, 16)
            r = nidx_v[sl]
            nlid_v[sl] = ((r >> 9) << 7) + (r & 127)
            nlane_v[sl] = ((r >> 7) & 3) << 5

        def round_body(src_hbm, lid_v, lane_v, out_hbm, out_stride, obase, t):
            pltpu.async_copy(
                src_hbm.at[lid_v.at[pl.ds(t * _R, _R)]], lines_v, sem
            ).wait()
            for c in range(_D):
                for g in range(_R // 16):
                    row16 = iota16 + (16 * g)
                    lane16 = lane_v[pl.ds(t * _R + 16 * g, 16)] + c
                    vals_v[c, pl.ds(16 * g, 16)] = plsc.load_gather(
                        lines_v, [row16, lane16])
            for c in range(_D):
                pltpu.sync_copy(
                    vals_v.at[c],
                    out_hbm.at[pl.ds(c * out_stride + obase + t * _R, _R)])

        for t in range(_BC // _R):
            round_body(lc_hbm, clid_v, clane_v, co_hbm, _B, cbase, t)

        @pl.loop(0, _BN // _R)
        def _(t):
            round_body(lx_hbm, nlid_v, nlane_v, no_hbm, _N, nbase, t)

    return gather_kernel(lines_c, lines_x, cidx, nidx)


def _log_sigmoid(x):
    return jnp.where(x >= 0, -jnp.log1p(jnp.exp(-x)), x - jnp.log1p(jnp.exp(x)))


def _tc_loss(pos2d, cep, nep):
    def loss_body(pos_ref, ce_ref, ne_ref, out_ref):
        pos_acc = jnp.sum(_log_sigmoid(pos_ref[...]))
        neg_acc = jnp.float32(0.0)
        rows_b = _B // 128
        rows_n = _N // 128
        for k in range(_K):
            s = jnp.zeros((rows_b, 128), jnp.float32)
            for c in range(_D):
                ce = ce_ref[pl.ds(c * rows_b, rows_b), :]
                ne = ne_ref[pl.ds(c * rows_n + k * rows_b, rows_b), :]
                s += ce * ne
            neg_acc += jnp.sum(_log_sigmoid(-s))
        out_ref[0, 0] = -pos_acc / _B - neg_acc / _N

    return pl.pallas_call(
        loss_body,
        out_shape=jax.ShapeDtypeStruct((1, 1), jnp.float32),
        out_specs=pl.BlockSpec(memory_space=pltpu.SMEM),
    )(pos2d, cep, nep)


def kernel(pos_scores, center_words, center_table, context_table):
    batch = pos_scores.shape[0]
    cidx = center_words.astype(jnp.int32)
    neg_words = jax.random.randint(
        jax.random.key(42), (batch, _K), 0, _VOCAB)
    nidx = neg_words.T.reshape(-1).astype(jnp.int32)
    lines_c = _linify(center_table.T)
    lines_x = _linify(context_table.T)
    ce_flat, ne_flat = _sc_gather(lines_c, lines_x, cidx, nidx)
    pos2d = pos_scores.reshape(128, 128)
    cep = ce_flat.reshape(_D * _B // 128, 128)
    nep = ne_flat.reshape(_D * _N // 128, 128)
    loss = _tc_loss(pos2d, cep, nep)
    return loss.reshape(())


# speedup vs baseline: 7.4835x; 1.3990x over previous
import functools

import jax
import jax.numpy as jnp
from jax import lax
from jax.experimental import pallas as pl
from jax.experimental.pallas import tpu as pltpu
from jax.experimental.pallas import tpu_sc as plsc

_VOCAB = 1_000_000
_D = 32
_K = 5
_NW = 32
_B = 16384
_N = _B * _K
_BC = _B // _NW
_BN = _N // _NW
_R = 256
_LW = 8192
_NBLK = (_VOCAB + _LW - 1) // _LW
_LINES = _NBLK * (_LW // 4)


def _linify(table_t):
    def body(x_ref, o_ref):
        x = x_ref[...]
        groups = []
        for g in range(_LW // 512):
            stacked = jnp.concatenate(
                [x[:, 512 * g + 128 * j:512 * g + 128 * (j + 1)]
                 for j in range(4)], axis=0)
            groups.append(stacked.T)
        o_ref[...] = jnp.concatenate(groups, axis=0)

    return pl.pallas_call(
        body,
        grid=(_NBLK,),
        in_specs=[pl.BlockSpec((_D, _LW), lambda i: (0, i))],
        out_specs=pl.BlockSpec((_LW // 4, 128), lambda i: (i, 0)),
        out_shape=jax.ShapeDtypeStruct((_LINES, 128), jnp.float32),
        compiler_params=pltpu.CompilerParams(
            dimension_semantics=("parallel",)),
    )(table_t)


def _sc_gather(lines, idx, total):
    per_w = total // _NW
    rounds = per_w // _R
    mesh = plsc.VectorSubcoreMesh(core_axis_name="c", subcore_axis_name="s")

    @functools.partial(
        pl.kernel,
        mesh=mesh,
        compiler_params=pltpu.CompilerParams(
            use_tc_tiling_on_sc=False, needs_layout_passes=False),
        out_type=jax.ShapeDtypeStruct((_D * total,), jnp.float32),
        scratch_types=[
            pltpu.VMEM((per_w,), jnp.int32),
            pltpu.VMEM((per_w,), jnp.int32),
            pltpu.VMEM((per_w,), jnp.int32),
            pltpu.VMEM((2, _R, 128), jnp.float32),
            pltpu.VMEM((_D, _R), jnp.float32),
            pltpu.SemaphoreType.DMA,
            pltpu.SemaphoreType.DMA,
        ],
    )
    def gather_kernel(l_hbm, i_hbm, o_hbm,
                      idx_v, lid_v, lane_v, lines_v, vals_v, sem0, sem1):
        wid = lax.axis_index("s") * 2 + lax.axis_index("c")
        iota16 = lax.iota(jnp.int32, 16)
        base = wid * per_w

        pltpu.sync_copy(i_hbm.at[pl.ds(base, per_w)], idx_v)
        for g in range(per_w // 16):
            sl = pl.ds(16 * g, 16)
            r = idx_v[sl]
            lid_v[sl] = ((r >> 9) << 7) + (r & 127)
            lane_v[sl] = ((r >> 7) & 3) << 5

        def fire(t, buf, sem):
            pltpu.async_copy(
                l_hbm.at[lid_v.at[pl.ds(t * _R, _R)]], lines_v.at[buf], sem)

        def wait_buf(buf, sem):
            pltpu.make_async_copy(
                l_hbm.at[lid_v.at[pl.ds(0, _R)]], lines_v.at[buf], sem).wait()

        def extract_and_store(t, buf):
            for c in range(_D):
                for g in range(_R // 16):
                    row16 = iota16 + (16 * g)
                    lane16 = lane_v[pl.ds(t * _R + 16 * g, 16)] + c
                    vals_v[c, pl.ds(16 * g, 16)] = plsc.load_gather(
                        lines_v.at[buf], [row16, lane16])
            for c in range(_D):
                pltpu.sync_copy(
                    vals_v.at[c],
                    o_hbm.at[pl.ds(c * total + base + t * _R, _R)])

        fire(0, 0, sem0)

        @pl.loop(0, rounds, step=2)
        def _(t):
            fire(t + 1, 1, sem1)
            wait_buf(0, sem0)
            extract_and_store(t, 0)

            @pl.when(t + 2 < rounds)
            def _():
                fire(t + 2, 0, sem0)

            wait_buf(1, sem1)
            extract_and_store(t + 1, 1)

    return gather_kernel(lines, idx)


def _log_sigmoid(x):
    return jnp.where(x >= 0, -jnp.log1p(jnp.exp(-x)), x - jnp.log1p(jnp.exp(x)))


def _tc_loss(pos2d, cep, nep):
    def loss_body(pos_ref, ce_ref, ne_ref, out_ref):
        pos_acc = jnp.sum(_log_sigmoid(pos_ref[...]))
        neg_acc = jnp.float32(0.0)
        rows_b = _B // 128
        rows_n = _N // 128
        for k in range(_K):
            s = jnp.zeros((rows_b, 128), jnp.float32)
            for c in range(_D):
                ce = ce_ref[pl.ds(c * rows_b, rows_b), :]
                ne = ne_ref[pl.ds(c * rows_n + k * rows_b, rows_b), :]
                s += ce * ne
            neg_acc += jnp.sum(_log_sigmoid(-s))
        out_ref[0, 0] = -pos_acc / _B - neg_acc / _N

    return pl.pallas_call(
        loss_body,
        out_shape=jax.ShapeDtypeStruct((1, 1), jnp.float32),
        out_specs=pl.BlockSpec(memory_space=pltpu.SMEM),
    )(pos2d, cep, nep)


def kernel(pos_scores, center_words, center_table, context_table):
    batch = pos_scores.shape[0]
    cidx = center_words.astype(jnp.int32)
    neg_words = jax.random.randint(
        jax.random.key(42), (batch, _K), 0, _VOCAB)
    nidx = neg_words.T.reshape(-1).astype(jnp.int32)
    lines_c = _linify(center_table.T)
    ce_flat = _sc_gather(lines_c, cidx, _B)
    lines_x = _linify(context_table.T)
    ne_flat = _sc_gather(lines_x, nidx, _N)
    pos2d = pos_scores.reshape(128, 128)
    cep = ce_flat.reshape(_D * _B // 128, 128)
    nep = ne_flat.reshape(_D * _N // 128, 128)
    loss = _tc_loss(pos2d, cep, nep)
    return loss.reshape(())
